# baseline (speedup 1.0000x reference)
import functools

import jax
import jax.numpy as jnp
from jax import lax
from jax.experimental import pallas as pl
from jax.experimental.pallas import tpu as pltpu
from jax.experimental.pallas import tpu_sc as plsc

NV = 100000
ND = 5
K = 1000
KP = 1024
C = 64
E = 4096
EB = 256


def _sc_gather(dataT_flat, clT_pad):
    info = plsc.get_sparse_core_info()
    nc, ns = info.num_cores, info.num_subcores
    nw = nc * ns
    ow = C // nw

    @functools.partial(
        pl.kernel,
        mesh=plsc.VectorSubcoreMesh(core_axis_name="c", subcore_axis_name="s"),
        out_type=jax.ShapeDtypeStruct((3, C, KP), jnp.float32),
        scratch_types=[
            pltpu.VMEM((ow, KP), jnp.int32),
            pltpu.VMEM((2, ow, KP), jnp.int32),
            pltpu.VMEM((3, ow, KP), jnp.float32),
            pltpu.SemaphoreType.DMA,
        ],
    )
    def body(data_hbm, cl_hbm, out_hbm, cid_v, idx_v, vout, sem):
        wid = lax.axis_index("s") * nc + lax.axis_index("c")
        i0 = wid * ow
        pltpu.sync_copy(cl_hbm.at[pl.ds(i0, ow), :], cid_v)
        for coord in (1, 2):
            for r in range(ow):
                for s in range(KP // 16):
                    sl = pl.ds(s * 16, 16)
                    idx_v[coord - 1, r, sl] = cid_v[r, sl] + coord * NV
        def idx_ref(coord, r, j):
            if coord == 0:
                return cid_v.at[r, pl.ds(j * 128, 128)]
            return idx_v.at[coord - 1, r, pl.ds(j * 128, 128)]

        copies = [
            pltpu.async_copy(
                data_hbm.at[idx_ref(coord, r, j)],
                vout.at[coord, r, pl.ds(j * 128, 128)], sem)
            for coord in range(3)
            for r in range(ow)
            for j in range(KP // 128)
        ]
        for cp in copies:
            cp.wait()
        pltpu.sync_copy(vout, out_hbm.at[:, pl.ds(i0, ow), :])

    return body(dataT_flat, clT_pad)


def _tc_body(cp_ref, e_ref, out_ref):
    ids = jnp.concatenate([e_ref[0, :], e_ref[1, :]])
    kio = lax.broadcasted_iota(jnp.int32, (KP, 2 * EB), 0)
    oh = (kio == ids[None, :]).astype(jnp.float32)
    x = lax.dot_general(
        cp_ref[...], oh,
        (((1,), (0,)), ((), ())),
        precision=lax.Precision.HIGHEST,
        preferred_element_type=jnp.float32,
    )
    x1x, x1y, x1z = x[0:C, :EB], x[C:2 * C, :EB], x[2 * C:3 * C, :EB]
    x2x, x2y, x2z = x[0:C, EB:], x[C:2 * C, EB:], x[2 * C:3 * C, EB:]

    dx = x1x[:, None, :] - x2x[None, :, :]
    dy = x1y[:, None, :] - x2y[None, :, :]
    dz = x1z[:, None, :] - x2z[None, :, :]
    d2 = dx * dx + dy * dy + dz * dz

    mrow = jnp.min(d2, axis=1)
    m = jnp.min(mrow, axis=0)
    cio = lax.broadcasted_iota(jnp.int32, (C, EB), 0)
    i1 = jnp.min(jnp.where(mrow == m[None, :], cio, C), axis=0)
    ohi = (cio == i1[None, :]).astype(jnp.float32)
    v1x = jnp.sum(x1x * ohi, axis=0)
    v1y = jnp.sum(x1y * ohi, axis=0)
    v1z = jnp.sum(x1z * ohi, axis=0)

    rx = v1x[None, :] - x2x
    ry = v1y[None, :] - x2y
    rz = v1z[None, :] - x2z
    drow = rx * rx + ry * ry + rz * rz
    j1 = jnp.min(jnp.where(drow == m[None, :], cio, C), axis=0)
    ohj = (cio == j1[None, :]).astype(jnp.float32)
    v2x = jnp.sum(x2x * ohj, axis=0)
    v2y = jnp.sum(x2y * ohj, axis=0)
    v2z = jnp.sum(x2z * ohj, axis=0)

    px = v1x - v2x
    py = v1y - v2y
    pz = v1z - v2z
    lend = jnp.sqrt(px * px + py * py + pz * pz)
    safe = jnp.maximum(lend, 1e-30)
    pos = lend > 0
    nx = jnp.where(pos, px / safe, px)
    ny = jnp.where(pos, py / safe, py)
    nz = jnp.where(pos, pz / safe, pz)

    rows = [v1x, v1y, v1z, v2x, v2y, v2z, nx, ny, nz, lend,
            nx * nx, nx * ny, nx * nz,
            ny * nx, ny * ny, ny * nz,
            nz * nx, nz * ny, nz * nz]
    out_ref[...] = jnp.stack(rows, axis=0)


def _tc_encode(cp, e32):
    return pl.pallas_call(
        _tc_body,
        grid=(E // EB,),
        in_specs=[
            pl.BlockSpec((3 * C, KP), lambda i: (0, 0)),
            pl.BlockSpec((2, EB), lambda i: (0, i)),
        ],
        out_specs=pl.BlockSpec((19, EB), lambda i: (0, i)),
        out_shape=jax.ShapeDtypeStruct((19, E), jnp.float32),
    )(cp, e32)


def kernel(data, clusts, edge_index):
    data = data.astype(jnp.float32)
    e32 = edge_index.astype(jnp.int32)
    dataT_flat = data.T[:3].reshape(-1)
    clT_pad = jnp.pad(clusts.astype(jnp.int32).T, ((0, 0), (0, KP - K)))
    cp = _sc_gather(dataT_flat, clT_pad).reshape(3 * C, KP)
    out = _tc_encode(cp, e32)
    return out.T


# speedup vs baseline: 1.0829x; 1.0829x over previous
import functools

import jax
import jax.numpy as jnp
from jax import lax
from jax.experimental import pallas as pl
from jax.experimental.pallas import tpu as pltpu
from jax.experimental.pallas import tpu_sc as plsc

NV = 100000
ND = 5
K = 1000
KP = 1024
C = 64
E = 4096
EB = 256


def _sc_gather(dataT_flat, clT_pad):
    info = plsc.get_sparse_core_info()
    nc, ns = info.num_cores, info.num_subcores
    nw = nc * ns
    ow = C // nw

    @functools.partial(
        pl.kernel,
        mesh=plsc.VectorSubcoreMesh(core_axis_name="c", subcore_axis_name="s"),
        out_type=jax.ShapeDtypeStruct((3, C, KP), jnp.float32),
        scratch_types=[
            pltpu.VMEM((ow, KP), jnp.int32),
            pltpu.VMEM((2, ow, KP), jnp.int32),
            pltpu.VMEM((3, ow, KP), jnp.float32),
            pltpu.SemaphoreType.DMA,
        ],
    )
    def body(data_hbm, cl_hbm, out_hbm, cid_v, idx_v, vout, sem):
        wid = lax.axis_index("s") * nc + lax.axis_index("c")
        i0 = wid * ow
        pltpu.sync_copy(cl_hbm.at[pl.ds(i0, ow), :], cid_v)
        for coord in (1, 2):
            for r in range(ow):
                for s in range(KP // 16):
                    sl = pl.ds(s * 16, 16)
                    idx_v[coord - 1, r, sl] = cid_v[r, sl] + coord * NV
        def idx_ref(coord, r, j):
            if coord == 0:
                return cid_v.at[r, pl.ds(j * 128, 128)]
            return idx_v.at[coord - 1, r, pl.ds(j * 128, 128)]

        copies = [
            pltpu.async_copy(
                data_hbm.at[idx_ref(coord, r, j)],
                vout.at[coord, r, pl.ds(j * 128, 128)], sem)
            for coord in range(3)
            for r in range(ow)
            for j in range(KP // 128)
        ]
        for cp in copies:
            cp.wait()
        pltpu.sync_copy(vout, out_hbm.at[:, pl.ds(i0, ow), :])

    return body(dataT_flat, clT_pad)


def _tc_body(cp_ref, e_ref, out_ref):
    ids = jnp.concatenate([e_ref[0, :], e_ref[1, :]])
    kio = lax.broadcasted_iota(jnp.int32, (KP, 2 * EB), 0)
    oh = (kio == ids[None, :]).astype(jnp.float32)
    x = lax.dot_general(
        cp_ref[...], oh,
        (((1,), (0,)), ((), ())),
        precision=lax.Precision.HIGHEST,
        preferred_element_type=jnp.float32,
    )
    x1x, x1y, x1z = x[0:C, :EB], x[C:2 * C, :EB], x[2 * C:3 * C, :EB]
    x2x, x2y, x2z = x[0:C, EB:], x[C:2 * C, EB:], x[2 * C:3 * C, EB:]

    IB = 8
    mrow_blocks = []
    for ib in range(C // IB):
        s = slice(ib * IB, (ib + 1) * IB)
        dx = x1x[s][:, None, :] - x2x[None, :, :]
        dy = x1y[s][:, None, :] - x2y[None, :, :]
        dz = x1z[s][:, None, :] - x2z[None, :, :]
        d2 = dx * dx + dy * dy + dz * dz
        mrow_blocks.append(jnp.min(d2, axis=1))
    mrow = jnp.concatenate(mrow_blocks, axis=0)
    m = jnp.min(mrow, axis=0)
    cio = lax.broadcasted_iota(jnp.int32, (C, EB), 0)
    i1 = jnp.min(jnp.where(mrow == m[None, :], cio, C), axis=0)
    ohi = (cio == i1[None, :]).astype(jnp.float32)
    v1x = jnp.sum(x1x * ohi, axis=0)
    v1y = jnp.sum(x1y * ohi, axis=0)
    v1z = jnp.sum(x1z * ohi, axis=0)

    rx = v1x[None, :] - x2x
    ry = v1y[None, :] - x2y
    rz = v1z[None, :] - x2z
    drow = rx * rx + ry * ry + rz * rz
    j1 = jnp.min(jnp.where(drow == m[None, :], cio, C), axis=0)
    ohj = (cio == j1[None, :]).astype(jnp.float32)
    v2x = jnp.sum(x2x * ohj, axis=0)
    v2y = jnp.sum(x2y * ohj, axis=0)
    v2z = jnp.sum(x2z * ohj, axis=0)

    px = v1x - v2x
    py = v1y - v2y
    pz = v1z - v2z
    lend = jnp.sqrt(px * px + py * py + pz * pz)
    safe = jnp.maximum(lend, 1e-30)
    pos = lend > 0
    nx = jnp.where(pos, px / safe, px)
    ny = jnp.where(pos, py / safe, py)
    nz = jnp.where(pos, pz / safe, pz)

    rows = [v1x, v1y, v1z, v2x, v2y, v2z, nx, ny, nz, lend,
            nx * nx, nx * ny, nx * nz,
            ny * nx, ny * ny, ny * nz,
            nz * nx, nz * ny, nz * nz]
    out_ref[...] = jnp.stack(rows, axis=0)


def _tc_encode(cp, e32):
    return pl.pallas_call(
        _tc_body,
        grid=(E // EB,),
        in_specs=[
            pl.BlockSpec((3 * C, KP), lambda i: (0, 0)),
            pl.BlockSpec((2, EB), lambda i: (0, i)),
        ],
        out_specs=pl.BlockSpec((19, EB), lambda i: (0, i)),
        out_shape=jax.ShapeDtypeStruct((19, E), jnp.float32),
    )(cp, e32)


def kernel(data, clusts, edge_index):
    data = data.astype(jnp.float32)
    e32 = edge_index.astype(jnp.int32)
    dataT_flat = data.T[:3].reshape(-1)
    clT_pad = jnp.pad(clusts.astype(jnp.int32).T, ((0, 0), (0, KP - K)))
    cp = _sc_gather(dataT_flat, clT_pad).reshape(3 * C, KP)
    out = _tc_encode(cp, e32)
    return out.T


# speedup vs baseline: 1.1892x; 1.0981x over previous
import functools

import jax
import jax.numpy as jnp
from jax import lax
from jax.experimental import pallas as pl
from jax.experimental.pallas import tpu as pltpu
from jax.experimental.pallas import tpu_sc as plsc

NV = 100000
ND = 5
K = 1000
KP = 1024
C = 64
E = 4096
EB = 256


def _sc_gather(dataT_flat, clT_pad):
    info = plsc.get_sparse_core_info()
    nc, ns = info.num_cores, info.num_subcores
    nw = nc * ns
    ow = C // nw

    @functools.partial(
        pl.kernel,
        mesh=plsc.VectorSubcoreMesh(core_axis_name="c", subcore_axis_name="s"),
        out_type=jax.ShapeDtypeStruct((3, C, KP), jnp.float32),
        scratch_types=[
            pltpu.VMEM((ow, KP), jnp.int32),
            pltpu.VMEM((2, ow, KP), jnp.int32),
            pltpu.VMEM((3, ow, KP), jnp.float32),
            pltpu.SemaphoreType.DMA,
        ],
    )
    def body(data_hbm, cl_hbm, out_hbm, cid_v, idx_v, vout, sem):
        wid = lax.axis_index("s") * nc + lax.axis_index("c")
        i0 = wid * ow
        pltpu.sync_copy(cl_hbm.at[pl.ds(i0, ow), :], cid_v)
        for coord in (1, 2):
            for r in range(ow):
                for s in range(KP // 16):
                    sl = pl.ds(s * 16, 16)
                    idx_v[coord - 1, r, sl] = cid_v[r, sl] + coord * NV
        def idx_ref(coord, r, j):
            if coord == 0:
                return cid_v.at[r, pl.ds(j * 128, 128)]
            return idx_v.at[coord - 1, r, pl.ds(j * 128, 128)]

        copies = [
            pltpu.async_copy(
                data_hbm.at[idx_ref(coord, r, j)],
                vout.at[coord, r, pl.ds(j * 128, 128)], sem)
            for coord in range(3)
            for r in range(ow)
            for j in range(KP // 128)
        ]
        for cp in copies:
            cp.wait()
        pltpu.sync_copy(vout, out_hbm.at[:, pl.ds(i0, ow), :])

    return body(dataT_flat, clT_pad)


def _tc_body(cp_ref, e_ref, out_ref):
    ids = jnp.concatenate([e_ref[0, :], e_ref[1, :]])
    kio = lax.broadcasted_iota(jnp.int32, (KP, 2 * EB), 0)
    oh = (kio == ids[None, :]).astype(jnp.float32)
    x = lax.dot_general(
        cp_ref[...], oh,
        (((1,), (0,)), ((), ())),
        precision=lax.Precision.HIGHEST,
        preferred_element_type=jnp.float32,
    )
    x1x, x1y, x1z = x[0:C, :EB], x[C:2 * C, :EB], x[2 * C:3 * C, :EB]
    x2x, x2y, x2z = x[0:C, EB:], x[C:2 * C, EB:], x[2 * C:3 * C, EB:]

    IB = 1
    mrow_blocks = []
    for ib in range(C // IB):
        s = slice(ib * IB, (ib + 1) * IB)
        dx = x1x[s][:, None, :] - x2x[None, :, :]
        dy = x1y[s][:, None, :] - x2y[None, :, :]
        dz = x1z[s][:, None, :] - x2z[None, :, :]
        d2 = dx * dx + dy * dy + dz * dz
        mrow_blocks.append(jnp.min(d2, axis=1))
    mrow = jnp.concatenate(mrow_blocks, axis=0)
    m = jnp.min(mrow, axis=0)
    cio = lax.broadcasted_iota(jnp.int32, (C, EB), 0)
    i1 = jnp.min(jnp.where(mrow == m[None, :], cio, C), axis=0)
    ohi = (cio == i1[None, :]).astype(jnp.float32)
    v1x = jnp.sum(x1x * ohi, axis=0)
    v1y = jnp.sum(x1y * ohi, axis=0)
    v1z = jnp.sum(x1z * ohi, axis=0)

    rx = v1x[None, :] - x2x
    ry = v1y[None, :] - x2y
    rz = v1z[None, :] - x2z
    drow = rx * rx + ry * ry + rz * rz
    j1 = jnp.min(jnp.where(drow == m[None, :], cio, C), axis=0)
    ohj = (cio == j1[None, :]).astype(jnp.float32)
    v2x = jnp.sum(x2x * ohj, axis=0)
    v2y = jnp.sum(x2y * ohj, axis=0)
    v2z = jnp.sum(x2z * ohj, axis=0)

    px = v1x - v2x
    py = v1y - v2y
    pz = v1z - v2z
    lend = jnp.sqrt(px * px + py * py + pz * pz)
    safe = jnp.maximum(lend, 1e-30)
    pos = lend > 0
    nx = jnp.where(pos, px / safe, px)
    ny = jnp.where(pos, py / safe, py)
    nz = jnp.where(pos, pz / safe, pz)

    rows = [v1x, v1y, v1z, v2x, v2y, v2z, nx, ny, nz, lend,
            nx * nx, nx * ny, nx * nz,
            ny * nx, ny * ny, ny * nz,
            nz * nx, nz * ny, nz * nz]
    out_ref[...] = jnp.stack(rows, axis=0)


def _tc_encode(cp, e32):
    return pl.pallas_call(
        _tc_body,
        grid=(E // EB,),
        in_specs=[
            pl.BlockSpec((3 * C, KP), lambda i: (0, 0)),
            pl.BlockSpec((2, EB), lambda i: (0, i)),
        ],
        out_specs=pl.BlockSpec((19, EB), lambda i: (0, i)),
        out_shape=jax.ShapeDtypeStruct((19, E), jnp.float32),
    )(cp, e32)


def kernel(data, clusts, edge_index):
    data = data.astype(jnp.float32)
    e32 = edge_index.astype(jnp.int32)
    dataT_flat = data.T[:3].reshape(-1)
    clT_pad = jnp.pad(clusts.astype(jnp.int32).T, ((0, 0), (0, KP - K)))
    cp = _sc_gather(dataT_flat, clT_pad).reshape(3 * C, KP)
    out = _tc_encode(cp, e32)
    return out.T
